# baseline (speedup 1.0000x reference)
import functools

import jax
import jax.numpy as jnp
from jax.experimental import pallas as pl
from jax.experimental.pallas import tpu as pltpu

T = 2048
D = 1024
FF = 2048
H = 16
KV = 8
HD = 64
E = 8
TOPK = 2
EPS = 1e-05
BASE = 1000000.0

BT = 256
BQ = 512
BM = 128
NPAD = 2 * T + E * BM
NB = NPAD // BM


def _qkv_kernel(x_ref, ln_ref, w_ref, cos_ref, sin_ref, o_ref):
    x = x_ref[...]
    var = jnp.mean(x * x, axis=-1, keepdims=True)
    h = x * jax.lax.rsqrt(var + EPS) * ln_ref[...]
    qkv = jax.lax.dot_general(h, w_ref[...], (((1,), (1,)), ((), ())),
                              preferred_element_type=jnp.float32)
    cos = cos_ref[...]
    sin = sin_ref[...]
    half = HD // 2
    for hd in range(H + KV):
        base = hd * HD
        x1 = qkv[:, base:base + half]
        x2 = qkv[:, base + half:base + HD]
        o_ref[hd, :, :half] = x1 * cos - x2 * sin
        o_ref[hd, :, half:] = x2 * cos + x1 * sin
    for hd in range(H + KV, H + 2 * KV):
        o_ref[hd, :, :] = qkv[:, hd * HD:(hd + 1) * HD]


def _attn_kernel(qoff, kl, q_ref, k_ref, v_ref, o_ref):
    q = q_ref[0]
    k = k_ref[0]
    s = jax.lax.dot_general(q, k, (((1,), (1,)), ((), ())),
                            preferred_element_type=jnp.float32)
    s = s * (HD ** -0.5)
    rows = qoff + jax.lax.broadcasted_iota(jnp.int32, (BQ, kl), 0)
    cols = jax.lax.broadcasted_iota(jnp.int32, (BQ, kl), 1)
    s = jnp.where(rows >= cols, s, -1e30)
    m = jnp.max(s, axis=-1, keepdims=True)
    p = jnp.exp(s - m)
    p = p / jnp.sum(p, axis=-1, keepdims=True)
    o = jax.lax.dot_general(p, v_ref[0], (((1,), (0,)), ((), ())),
                            preferred_element_type=jnp.float32)
    o_ref[...] = o.T


def _post_kernel(o_ref, res_ref, ow_ref, ln_ref, gw_ref,
                 hs_ref, h2_ref, comb_ref):
    attn_out = jax.lax.dot_general(o_ref[...], ow_ref[...],
                                   (((0,), (1,)), ((), ())),
                                   preferred_element_type=jnp.float32)
    hs = res_ref[...] + attn_out
    hs_ref[...] = hs
    var = jnp.mean(hs * hs, axis=-1, keepdims=True)
    h2 = hs * jax.lax.rsqrt(var + EPS) * ln_ref[...]
    h2_ref[...] = h2
    logits = jax.lax.dot_general(h2, gw_ref[...], (((1,), (1,)), ((), ())),
                                 preferred_element_type=jnp.float32)
    lmax = jnp.max(logits, axis=-1, keepdims=True)
    p = jnp.exp(logits - lmax)
    p = p / jnp.sum(p, axis=-1, keepdims=True)
    iota = jax.lax.broadcasted_iota(jnp.int32, (BT, E), 1)
    m1 = jnp.max(p, axis=-1, keepdims=True)
    i1 = jnp.min(jnp.where(p == m1, iota, E), axis=-1, keepdims=True)
    p2 = jnp.where(iota == i1, -1.0, p)
    m2 = jnp.max(p2, axis=-1, keepdims=True)
    i2 = jnp.min(jnp.where(p2 == m2, iota, E), axis=-1, keepdims=True)
    denom = m1 + m2
    comb = jnp.where(iota == i1, m1 / denom, 0.0)
    comb = jnp.where(iota == i2, m2 / denom, comb)
    comb_ref[...] = comb


def _moe_up_kernel(tok_ref, be_ref, h2_ref, w1_ref, w3_ref, hh_ref, x_s):
    b = pl.program_id(0)
    base = b * BM

    def gather(i, _):
        t = tok_ref[base + i]
        x_s[pl.ds(i, 1), :] = h2_ref[pl.ds(t, 1), :]
        return 0

    jax.lax.fori_loop(0, BM, gather, 0, unroll=8)

    x = x_s[...]
    h1 = jax.lax.dot_general(x, w1_ref[0], (((1,), (1,)), ((), ())),
                             preferred_element_type=jnp.float32)
    h3 = jax.lax.dot_general(x, w3_ref[0], (((1,), (1,)), ((), ())),
                             preferred_element_type=jnp.float32)
    hh_ref[...] = (h1 * jax.nn.sigmoid(h1)) * h3


def _moe_down_kernel(tok_ref, wgt_ref, be_ref, hh_ref, hs_ref,
                     w2_ref, out_ref, acc_s):
    b = pl.program_id(0)

    @pl.when(b == 0)
    def _init():
        out_ref[...] = hs_ref[...]

    base = b * BM
    acc_s[...] = jax.lax.dot_general(hh_ref[...], w2_ref[0],
                                     (((1,), (1,)), ((), ())),
                                     preferred_element_type=jnp.float32)

    def scatter(i, _):
        t = tok_ref[base + i]
        w = wgt_ref[base + i]
        row = acc_s[pl.ds(i, 1), :]
        out_ref[pl.ds(t, 1), :] = out_ref[pl.ds(t, 1), :] + w * row
        return 0

    jax.lax.fori_loop(0, BM, scatter, 0, unroll=8)


def kernel(positions, hidden_states, ln1_w, qkv_w, o_w, ln2_w, gate_w,
           w1, w2, w3):
    f32 = jnp.float32
    inv_freq = 1.0 / (BASE ** (jnp.arange(0, HD, 2, dtype=f32) / HD))
    f = positions.astype(f32)[:, None] * inv_freq
    cos = jnp.cos(f)
    sin = jnp.sin(f)
    ln1 = ln1_w.reshape(1, D)
    ln2 = ln2_w.reshape(1, D)

    qkv = pl.pallas_call(
        _qkv_kernel,
        grid=(T // BT,),
        in_specs=[
            pl.BlockSpec((BT, D), lambda i: (i, 0)),
            pl.BlockSpec((1, D), lambda i: (0, 0)),
            pl.BlockSpec(((H + 2 * KV) * HD, D), lambda i: (0, 0)),
            pl.BlockSpec((BT, HD // 2), lambda i: (i, 0)),
            pl.BlockSpec((BT, HD // 2), lambda i: (i, 0)),
        ],
        out_specs=pl.BlockSpec((H + 2 * KV, BT, HD), lambda i: (0, i, 0)),
        out_shape=jax.ShapeDtypeStruct((H + 2 * KV, T, HD), f32),
    )(hidden_states, ln1, qkv_w, cos, sin)

    rep = H // KV
    o_parts = []
    for r in range(T // BQ):
        kl = (r + 1) * BQ
        o_parts.append(pl.pallas_call(
            functools.partial(_attn_kernel, r * BQ, kl),
            grid=(H,),
            in_specs=[
                pl.BlockSpec((1, BQ, HD), lambda h, r=r: (h, r, 0)),
                pl.BlockSpec((1, kl, HD), lambda h: (H + h // rep, 0, 0)),
                pl.BlockSpec((1, kl, HD), lambda h: (H + KV + h // rep, 0, 0)),
            ],
            out_specs=pl.BlockSpec((HD, BQ), lambda h: (h, 0)),
            out_shape=jax.ShapeDtypeStruct((H * HD, BQ), f32),
        )(qkv, qkv, qkv))
    o = jnp.concatenate(o_parts, axis=1)

    hs, h2, comb = pl.pallas_call(
        _post_kernel,
        grid=(T // BT,),
        in_specs=[
            pl.BlockSpec((H * HD, BT), lambda i: (0, i)),
            pl.BlockSpec((BT, D), lambda i: (i, 0)),
            pl.BlockSpec((D, H * HD), lambda i: (0, 0)),
            pl.BlockSpec((1, D), lambda i: (0, 0)),
            pl.BlockSpec((E, D), lambda i: (0, 0)),
        ],
        out_specs=[
            pl.BlockSpec((BT, D), lambda i: (i, 0)),
            pl.BlockSpec((BT, D), lambda i: (i, 0)),
            pl.BlockSpec((BT, E), lambda i: (i, 0)),
        ],
        out_shape=[
            jax.ShapeDtypeStruct((T, D), f32),
            jax.ShapeDtypeStruct((T, D), f32),
            jax.ShapeDtypeStruct((T, E), f32),
        ],
    )(o, hidden_states, o_w, ln2, gate_w)

    ohi = (comb > 0).astype(jnp.int32)
    cum = jnp.cumsum(ohi, axis=0)
    counts = cum[-1]
    rank = cum - ohi
    pcounts = ((counts + BM - 1) // BM) * BM
    poff = jnp.concatenate([jnp.zeros((1,), jnp.int32),
                            jnp.cumsum(pcounts).astype(jnp.int32)])
    pos = poff[None, :E] + rank
    posv = jnp.where(ohi > 0, pos, NPAD).reshape(-1)
    tok_b = jnp.broadcast_to(jnp.arange(T, dtype=jnp.int32)[:, None],
                             (T, E)).reshape(-1)
    dst_tok = jnp.zeros((NPAD,), jnp.int32).at[posv].set(tok_b, mode='drop')
    dst_w = jnp.zeros((NPAD,), f32).at[posv].set(comb.reshape(-1),
                                                 mode='drop')
    starts = jnp.arange(NB, dtype=jnp.int32) * BM
    block_expert = jnp.minimum(
        jnp.sum((starts[:, None] >= poff[None, 1:]).astype(jnp.int32),
                axis=1), E - 1).astype(jnp.int32)

    up_spec = pltpu.PrefetchScalarGridSpec(
        num_scalar_prefetch=2,
        grid=(NB,),
        in_specs=[
            pl.BlockSpec((T, D), lambda b, tok, be: (0, 0)),
            pl.BlockSpec((1, FF, D), lambda b, tok, be: (be[b], 0, 0)),
            pl.BlockSpec((1, FF, D), lambda b, tok, be: (be[b], 0, 0)),
        ],
        out_specs=pl.BlockSpec((BM, FF), lambda b, tok, be: (b, 0)),
        scratch_shapes=[pltpu.VMEM((BM, D), f32)],
    )
    hh = pl.pallas_call(
        _moe_up_kernel,
        grid_spec=up_spec,
        out_shape=jax.ShapeDtypeStruct((NPAD, FF), f32),
    )(dst_tok, block_expert, h2, w1, w3)

    down_spec = pltpu.PrefetchScalarGridSpec(
        num_scalar_prefetch=3,
        grid=(NB,),
        in_specs=[
            pl.BlockSpec((BM, FF), lambda b, tok, wgt, be: (b, 0)),
            pl.BlockSpec((T, D), lambda b, tok, wgt, be: (0, 0)),
            pl.BlockSpec((1, D, FF), lambda b, tok, wgt, be: (be[b], 0, 0)),
        ],
        out_specs=pl.BlockSpec((T, D), lambda b, tok, wgt, be: (0, 0)),
        scratch_shapes=[pltpu.VMEM((BM, D), f32)],
    )
    out = pl.pallas_call(
        _moe_down_kernel,
        grid_spec=down_spec,
        out_shape=jax.ShapeDtypeStruct((T, D), f32),
    )(dst_tok, dst_w, block_expert, hh, hs, w2)

    return out


# speedup vs baseline: 1.0387x; 1.0387x over previous
import functools

import jax
import jax.numpy as jnp
from jax.experimental import pallas as pl
from jax.experimental.pallas import tpu as pltpu

T = 2048
D = 1024
FF = 2048
H = 16
KV = 8
HD = 64
E = 8
TOPK = 2
EPS = 1e-05
BASE = 1000000.0

BT = 256
BQ = 512
BM = 128
NPAD = 2 * T + E * BM
NB = NPAD // BM


def _qkv_kernel(x_ref, ln_ref, w_ref, cos_ref, sin_ref, o_ref):
    x = x_ref[...]
    var = jnp.mean(x * x, axis=-1, keepdims=True)
    h = x * jax.lax.rsqrt(var + EPS) * ln_ref[...]
    qkv = jax.lax.dot_general(h, w_ref[...], (((1,), (1,)), ((), ())),
                              preferred_element_type=jnp.float32)
    cos = cos_ref[...]
    sin = sin_ref[...]
    half = HD // 2
    for hd in range(H + KV):
        base = hd * HD
        x1 = qkv[:, base:base + half]
        x2 = qkv[:, base + half:base + HD]
        o_ref[hd, :, :half] = x1 * cos - x2 * sin
        o_ref[hd, :, half:] = x2 * cos + x1 * sin
    for hd in range(H + KV, H + 2 * KV):
        o_ref[hd, :, :] = qkv[:, hd * HD:(hd + 1) * HD]


def _attn_kernel(qoff, kl, q_ref, k_ref, v_ref, o_ref):
    q = q_ref[0]
    k = k_ref[0]
    s = jax.lax.dot_general(q, k, (((1,), (1,)), ((), ())),
                            preferred_element_type=jnp.float32)
    s = s * (HD ** -0.5)
    rows = qoff + jax.lax.broadcasted_iota(jnp.int32, (BQ, kl), 0)
    cols = jax.lax.broadcasted_iota(jnp.int32, (BQ, kl), 1)
    s = jnp.where(rows >= cols, s, -1e30)
    m = jnp.max(s, axis=-1, keepdims=True)
    p = jnp.exp(s - m)
    p = p / jnp.sum(p, axis=-1, keepdims=True)
    o = jax.lax.dot_general(p, v_ref[0], (((1,), (0,)), ((), ())),
                            preferred_element_type=jnp.float32)
    o_ref[...] = o.T


def _post_kernel(o_ref, res_ref, ow_ref, ln_ref, gw_ref,
                 hs_ref, h2_ref, idx_ref, wt_ref):
    attn_out = jax.lax.dot_general(o_ref[...], ow_ref[...],
                                   (((0,), (1,)), ((), ())),
                                   preferred_element_type=jnp.float32)
    hs = res_ref[...] + attn_out
    hs_ref[...] = hs
    var = jnp.mean(hs * hs, axis=-1, keepdims=True)
    h2 = hs * jax.lax.rsqrt(var + EPS) * ln_ref[...]
    h2_ref[...] = h2
    logits = jax.lax.dot_general(h2, gw_ref[...], (((1,), (1,)), ((), ())),
                                 preferred_element_type=jnp.float32)
    lmax = jnp.max(logits, axis=-1, keepdims=True)
    p = jnp.exp(logits - lmax)
    p = p / jnp.sum(p, axis=-1, keepdims=True)
    iota = jax.lax.broadcasted_iota(jnp.int32, (BT, E), 1)
    m1 = jnp.max(p, axis=-1, keepdims=True)
    i1 = jnp.min(jnp.where(p == m1, iota, E), axis=-1, keepdims=True)
    p2 = jnp.where(iota == i1, -1.0, p)
    m2 = jnp.max(p2, axis=-1, keepdims=True)
    i2 = jnp.min(jnp.where(p2 == m2, iota, E), axis=-1, keepdims=True)
    denom = m1 + m2
    idx_ref[...] = jnp.where(iota == 0, i1, jnp.where(iota == 1, i2, 0))
    wt_ref[...] = jnp.where(iota == 0, m1 / denom,
                            jnp.where(iota == 1, m2 / denom, 0.0))


def _moe_up_kernel(tok_ref, be_ref, h2_ref, w1_ref, w3_ref, hh_ref, x_s):
    b = pl.program_id(0)
    base = b * BM

    def gather(i, _):
        t = tok_ref[base + i]
        x_s[pl.ds(i, 1), :] = h2_ref[pl.ds(t, 1), :]
        return 0

    jax.lax.fori_loop(0, BM, gather, 0, unroll=8)

    x = x_s[...]
    h1 = jax.lax.dot_general(x, w1_ref[0], (((1,), (1,)), ((), ())),
                             preferred_element_type=jnp.float32)
    h3 = jax.lax.dot_general(x, w3_ref[0], (((1,), (1,)), ((), ())),
                             preferred_element_type=jnp.float32)
    hh_ref[...] = (h1 * jax.nn.sigmoid(h1)) * h3


def _moe_down_kernel(tok_ref, wgt_ref, be_ref, hh_ref, hs_ref,
                     w2_ref, out_ref, acc_s):
    b = pl.program_id(0)

    @pl.when(b == 0)
    def _init():
        out_ref[...] = hs_ref[...]

    base = b * BM
    acc_s[...] = jax.lax.dot_general(hh_ref[...], w2_ref[0],
                                     (((1,), (1,)), ((), ())),
                                     preferred_element_type=jnp.float32)

    def scatter(i, _):
        t = tok_ref[base + i]
        w = wgt_ref[base + i]
        row = acc_s[pl.ds(i, 1), :]
        out_ref[pl.ds(t, 1), :] = out_ref[pl.ds(t, 1), :] + w * row
        return 0

    jax.lax.fori_loop(0, BM, scatter, 0, unroll=8)


def kernel(positions, hidden_states, ln1_w, qkv_w, o_w, ln2_w, gate_w,
           w1, w2, w3):
    f32 = jnp.float32
    inv_freq = 1.0 / (BASE ** (jnp.arange(0, HD, 2, dtype=f32) / HD))
    f = positions.astype(f32)[:, None] * inv_freq
    cos = jnp.cos(f)
    sin = jnp.sin(f)
    ln1 = ln1_w.reshape(1, D)
    ln2 = ln2_w.reshape(1, D)

    qkv = pl.pallas_call(
        _qkv_kernel,
        grid=(T // BT,),
        in_specs=[
            pl.BlockSpec((BT, D), lambda i: (i, 0)),
            pl.BlockSpec((1, D), lambda i: (0, 0)),
            pl.BlockSpec(((H + 2 * KV) * HD, D), lambda i: (0, 0)),
            pl.BlockSpec((BT, HD // 2), lambda i: (i, 0)),
            pl.BlockSpec((BT, HD // 2), lambda i: (i, 0)),
        ],
        out_specs=pl.BlockSpec((H + 2 * KV, BT, HD), lambda i: (0, i, 0)),
        out_shape=jax.ShapeDtypeStruct((H + 2 * KV, T, HD), f32),
    )(hidden_states, ln1, qkv_w, cos, sin)

    rep = H // KV
    o_parts = []
    for r in range(T // BQ):
        kl = (r + 1) * BQ
        o_parts.append(pl.pallas_call(
            functools.partial(_attn_kernel, r * BQ, kl),
            grid=(H,),
            in_specs=[
                pl.BlockSpec((1, BQ, HD), lambda h, r=r: (h, r, 0)),
                pl.BlockSpec((1, kl, HD), lambda h: (H + h // rep, 0, 0)),
                pl.BlockSpec((1, kl, HD), lambda h: (H + KV + h // rep, 0, 0)),
            ],
            out_specs=pl.BlockSpec((HD, BQ), lambda h: (h, 0)),
            out_shape=jax.ShapeDtypeStruct((H * HD, BQ), f32),
        )(qkv, qkv, qkv))
    o = jnp.concatenate(o_parts, axis=1)

    hs, h2, idx2, wt2 = pl.pallas_call(
        _post_kernel,
        grid=(T // BT,),
        in_specs=[
            pl.BlockSpec((H * HD, BT), lambda i: (0, i)),
            pl.BlockSpec((BT, D), lambda i: (i, 0)),
            pl.BlockSpec((D, H * HD), lambda i: (0, 0)),
            pl.BlockSpec((1, D), lambda i: (0, 0)),
            pl.BlockSpec((E, D), lambda i: (0, 0)),
        ],
        out_specs=[
            pl.BlockSpec((BT, D), lambda i: (i, 0)),
            pl.BlockSpec((BT, D), lambda i: (i, 0)),
            pl.BlockSpec((BT, E), lambda i: (i, 0)),
            pl.BlockSpec((BT, E), lambda i: (i, 0)),
        ],
        out_shape=[
            jax.ShapeDtypeStruct((T, D), f32),
            jax.ShapeDtypeStruct((T, D), f32),
            jax.ShapeDtypeStruct((T, E), jnp.int32),
            jax.ShapeDtypeStruct((T, E), f32),
        ],
    )(o, hidden_states, o_w, ln2, gate_w)

    eflat = idx2[:, :TOPK].reshape(-1)
    tflat = jnp.repeat(jnp.arange(T, dtype=jnp.int32), TOPK)
    wflat = wt2[:, :TOPK].reshape(-1)
    order = jnp.argsort(eflat)
    es = eflat[order]
    ts = tflat[order]
    ws = wflat[order]
    counts = jnp.bincount(eflat, length=E)
    pcounts = ((counts + BM - 1) // BM) * BM
    poff = jnp.concatenate([jnp.zeros((1,), jnp.int32),
                            jnp.cumsum(pcounts).astype(jnp.int32)])
    roff = jnp.concatenate([jnp.zeros((1,), jnp.int32),
                            jnp.cumsum(counts).astype(jnp.int32)])
    pos = poff[es] + (jnp.arange(2 * T, dtype=jnp.int32) - roff[es])
    dst_tok = jnp.zeros((NPAD,), jnp.int32).at[pos].set(ts)
    dst_w = jnp.zeros((NPAD,), f32).at[pos].set(ws)
    block_expert = jnp.clip(
        jnp.searchsorted(poff[1:], jnp.arange(NB, dtype=jnp.int32) * BM,
                         side='right'), 0, E - 1).astype(jnp.int32)

    up_spec = pltpu.PrefetchScalarGridSpec(
        num_scalar_prefetch=2,
        grid=(NB,),
        in_specs=[
            pl.BlockSpec((T, D), lambda b, tok, be: (0, 0)),
            pl.BlockSpec((1, FF, D), lambda b, tok, be: (be[b], 0, 0)),
            pl.BlockSpec((1, FF, D), lambda b, tok, be: (be[b], 0, 0)),
        ],
        out_specs=pl.BlockSpec((BM, FF), lambda b, tok, be: (b, 0)),
        scratch_shapes=[pltpu.VMEM((BM, D), f32)],
    )
    hh = pl.pallas_call(
        _moe_up_kernel,
        grid_spec=up_spec,
        out_shape=jax.ShapeDtypeStruct((NPAD, FF), f32),
    )(dst_tok, block_expert, h2, w1, w3)

    down_spec = pltpu.PrefetchScalarGridSpec(
        num_scalar_prefetch=3,
        grid=(NB,),
        in_specs=[
            pl.BlockSpec((BM, FF), lambda b, tok, wgt, be: (b, 0)),
            pl.BlockSpec((T, D), lambda b, tok, wgt, be: (0, 0)),
            pl.BlockSpec((1, D, FF), lambda b, tok, wgt, be: (be[b], 0, 0)),
        ],
        out_specs=pl.BlockSpec((T, D), lambda b, tok, wgt, be: (0, 0)),
        scratch_shapes=[pltpu.VMEM((BM, D), f32)],
    )
    out = pl.pallas_call(
        _moe_down_kernel,
        grid_spec=down_spec,
        out_shape=jax.ShapeDtypeStruct((T, D), f32),
    )(dst_tok, dst_w, block_expert, hh, hs, w2)

    return out


# speedup vs baseline: 1.0501x; 1.0110x over previous
import functools

import jax
import jax.numpy as jnp
from jax.experimental import pallas as pl
from jax.experimental.pallas import tpu as pltpu

T = 2048
D = 1024
FF = 2048
H = 16
KV = 8
HD = 64
E = 8
TOPK = 2
EPS = 1e-05
BASE = 1000000.0

BT = 256
BQ = 1024
BM = 128
NPAD = 2 * T + E * BM
NB = NPAD // BM


def _qkv_kernel(x_ref, ln_ref, w_ref, cos_ref, sin_ref, o_ref):
    x = x_ref[...]
    var = jnp.mean(x * x, axis=-1, keepdims=True)
    h = x * jax.lax.rsqrt(var + EPS) * ln_ref[...]
    qkv = jax.lax.dot_general(h, w_ref[...], (((1,), (1,)), ((), ())),
                              preferred_element_type=jnp.float32)
    cos = cos_ref[...]
    sin = sin_ref[...]
    half = HD // 2
    for hd in range(H + KV):
        base = hd * HD
        x1 = qkv[:, base:base + half]
        x2 = qkv[:, base + half:base + HD]
        o_ref[hd, :, :half] = x1 * cos - x2 * sin
        o_ref[hd, :, half:] = x2 * cos + x1 * sin
    for hd in range(H + KV, H + 2 * KV):
        o_ref[hd, :, :] = qkv[:, hd * HD:(hd + 1) * HD]


def _attn_kernel(qoff, kl, q_ref, k_ref, v_ref, o_ref):
    q = q_ref[0]
    k = k_ref[0]
    s = jax.lax.dot_general(q, k, (((1,), (1,)), ((), ())),
                            preferred_element_type=jnp.float32)
    s = s * (HD ** -0.5)
    rows = qoff + jax.lax.broadcasted_iota(jnp.int32, (BQ, kl), 0)
    cols = jax.lax.broadcasted_iota(jnp.int32, (BQ, kl), 1)
    s = jnp.where(rows >= cols, s, -1e30)
    m = jnp.max(s, axis=-1, keepdims=True)
    p = jnp.exp(s - m)
    p = p / jnp.sum(p, axis=-1, keepdims=True)
    o = jax.lax.dot_general(p, v_ref[0], (((1,), (0,)), ((), ())),
                            preferred_element_type=jnp.float32)
    o_ref[...] = o.T


def _post_kernel(o_ref, res_ref, ow_ref, ln_ref, gw_ref,
                 hs_ref, h2_ref, idx_ref, wt_ref):
    attn_out = jax.lax.dot_general(o_ref[...], ow_ref[...],
                                   (((0,), (1,)), ((), ())),
                                   preferred_element_type=jnp.float32)
    hs = res_ref[...] + attn_out
    hs_ref[...] = hs
    var = jnp.mean(hs * hs, axis=-1, keepdims=True)
    h2 = hs * jax.lax.rsqrt(var + EPS) * ln_ref[...]
    h2_ref[...] = h2
    logits = jax.lax.dot_general(h2, gw_ref[...], (((1,), (1,)), ((), ())),
                                 preferred_element_type=jnp.float32)
    lmax = jnp.max(logits, axis=-1, keepdims=True)
    p = jnp.exp(logits - lmax)
    p = p / jnp.sum(p, axis=-1, keepdims=True)
    iota = jax.lax.broadcasted_iota(jnp.int32, (BT, E), 1)
    m1 = jnp.max(p, axis=-1, keepdims=True)
    i1 = jnp.min(jnp.where(p == m1, iota, E), axis=-1, keepdims=True)
    p2 = jnp.where(iota == i1, -1.0, p)
    m2 = jnp.max(p2, axis=-1, keepdims=True)
    i2 = jnp.min(jnp.where(p2 == m2, iota, E), axis=-1, keepdims=True)
    denom = m1 + m2
    idx_ref[...] = jnp.where(iota == 0, i1, jnp.where(iota == 1, i2, 0))
    wt_ref[...] = jnp.where(iota == 0, m1 / denom,
                            jnp.where(iota == 1, m2 / denom, 0.0))


def _moe_up_kernel(tok_ref, be_ref, h2_ref, w1_ref, w3_ref, hh_ref, x_s):
    b = pl.program_id(0)
    base = b * BM

    def gather(i, _):
        t = tok_ref[base + i]
        x_s[pl.ds(i, 1), :] = h2_ref[pl.ds(t, 1), :]
        return 0

    jax.lax.fori_loop(0, BM, gather, 0, unroll=8)

    x = x_s[...]
    h1 = jax.lax.dot_general(x, w1_ref[0], (((1,), (1,)), ((), ())),
                             preferred_element_type=jnp.float32)
    h3 = jax.lax.dot_general(x, w3_ref[0], (((1,), (1,)), ((), ())),
                             preferred_element_type=jnp.float32)
    hh_ref[...] = (h1 * jax.nn.sigmoid(h1)) * h3


def _moe_down_kernel(tok_ref, wgt_ref, be_ref, hh_ref, hs_ref,
                     w2_ref, out_ref, acc_s):
    b = pl.program_id(0)

    @pl.when(b == 0)
    def _init():
        out_ref[...] = hs_ref[...]

    base = b * BM
    acc_s[...] = jax.lax.dot_general(hh_ref[...], w2_ref[0],
                                     (((1,), (1,)), ((), ())),
                                     preferred_element_type=jnp.float32)

    def scatter(i, _):
        t = tok_ref[base + i]
        w = wgt_ref[base + i]
        row = acc_s[pl.ds(i, 1), :]
        out_ref[pl.ds(t, 1), :] = out_ref[pl.ds(t, 1), :] + w * row
        return 0

    jax.lax.fori_loop(0, BM, scatter, 0, unroll=8)


def kernel(positions, hidden_states, ln1_w, qkv_w, o_w, ln2_w, gate_w,
           w1, w2, w3):
    f32 = jnp.float32
    inv_freq = 1.0 / (BASE ** (jnp.arange(0, HD, 2, dtype=f32) / HD))
    f = positions.astype(f32)[:, None] * inv_freq
    cos = jnp.cos(f)
    sin = jnp.sin(f)
    ln1 = ln1_w.reshape(1, D)
    ln2 = ln2_w.reshape(1, D)

    qkv = pl.pallas_call(
        _qkv_kernel,
        grid=(T // BT,),
        in_specs=[
            pl.BlockSpec((BT, D), lambda i: (i, 0)),
            pl.BlockSpec((1, D), lambda i: (0, 0)),
            pl.BlockSpec(((H + 2 * KV) * HD, D), lambda i: (0, 0)),
            pl.BlockSpec((BT, HD // 2), lambda i: (i, 0)),
            pl.BlockSpec((BT, HD // 2), lambda i: (i, 0)),
        ],
        out_specs=pl.BlockSpec((H + 2 * KV, BT, HD), lambda i: (0, i, 0)),
        out_shape=jax.ShapeDtypeStruct((H + 2 * KV, T, HD), f32),
    )(hidden_states, ln1, qkv_w, cos, sin)

    rep = H // KV
    o_parts = []
    for r in range(T // BQ):
        kl = (r + 1) * BQ
        o_parts.append(pl.pallas_call(
            functools.partial(_attn_kernel, r * BQ, kl),
            grid=(H,),
            in_specs=[
                pl.BlockSpec((1, BQ, HD), lambda h, r=r: (h, r, 0)),
                pl.BlockSpec((1, kl, HD), lambda h: (H + h // rep, 0, 0)),
                pl.BlockSpec((1, kl, HD), lambda h: (H + KV + h // rep, 0, 0)),
            ],
            out_specs=pl.BlockSpec((HD, BQ), lambda h: (h, 0)),
            out_shape=jax.ShapeDtypeStruct((H * HD, BQ), f32),
        )(qkv, qkv, qkv))
    o = jnp.concatenate(o_parts, axis=1)

    hs, h2, idx2, wt2 = pl.pallas_call(
        _post_kernel,
        grid=(T // BT,),
        in_specs=[
            pl.BlockSpec((H * HD, BT), lambda i: (0, i)),
            pl.BlockSpec((BT, D), lambda i: (i, 0)),
            pl.BlockSpec((D, H * HD), lambda i: (0, 0)),
            pl.BlockSpec((1, D), lambda i: (0, 0)),
            pl.BlockSpec((E, D), lambda i: (0, 0)),
        ],
        out_specs=[
            pl.BlockSpec((BT, D), lambda i: (i, 0)),
            pl.BlockSpec((BT, D), lambda i: (i, 0)),
            pl.BlockSpec((BT, E), lambda i: (i, 0)),
            pl.BlockSpec((BT, E), lambda i: (i, 0)),
        ],
        out_shape=[
            jax.ShapeDtypeStruct((T, D), f32),
            jax.ShapeDtypeStruct((T, D), f32),
            jax.ShapeDtypeStruct((T, E), jnp.int32),
            jax.ShapeDtypeStruct((T, E), f32),
        ],
    )(o, hidden_states, o_w, ln2, gate_w)

    eflat = idx2[:, :TOPK].reshape(-1)
    tflat = jnp.repeat(jnp.arange(T, dtype=jnp.int32), TOPK)
    wflat = wt2[:, :TOPK].reshape(-1)
    order = jnp.argsort(eflat)
    es = eflat[order]
    ts = tflat[order]
    ws = wflat[order]
    counts = jnp.bincount(eflat, length=E)
    pcounts = ((counts + BM - 1) // BM) * BM
    poff = jnp.concatenate([jnp.zeros((1,), jnp.int32),
                            jnp.cumsum(pcounts).astype(jnp.int32)])
    roff = jnp.concatenate([jnp.zeros((1,), jnp.int32),
                            jnp.cumsum(counts).astype(jnp.int32)])
    pos = poff[es] + (jnp.arange(2 * T, dtype=jnp.int32) - roff[es])
    dst_tok = jnp.zeros((NPAD,), jnp.int32).at[pos].set(ts)
    dst_w = jnp.zeros((NPAD,), f32).at[pos].set(ws)
    block_expert = jnp.clip(
        jnp.searchsorted(poff[1:], jnp.arange(NB, dtype=jnp.int32) * BM,
                         side='right'), 0, E - 1).astype(jnp.int32)

    up_spec = pltpu.PrefetchScalarGridSpec(
        num_scalar_prefetch=2,
        grid=(NB,),
        in_specs=[
            pl.BlockSpec((T, D), lambda b, tok, be: (0, 0)),
            pl.BlockSpec((1, FF, D), lambda b, tok, be: (be[b], 0, 0)),
            pl.BlockSpec((1, FF, D), lambda b, tok, be: (be[b], 0, 0)),
        ],
        out_specs=pl.BlockSpec((BM, FF), lambda b, tok, be: (b, 0)),
        scratch_shapes=[pltpu.VMEM((BM, D), f32)],
    )
    hh = pl.pallas_call(
        _moe_up_kernel,
        grid_spec=up_spec,
        out_shape=jax.ShapeDtypeStruct((NPAD, FF), f32),
    )(dst_tok, block_expert, h2, w1, w3)

    down_spec = pltpu.PrefetchScalarGridSpec(
        num_scalar_prefetch=3,
        grid=(NB,),
        in_specs=[
            pl.BlockSpec((BM, FF), lambda b, tok, wgt, be: (b, 0)),
            pl.BlockSpec((T, D), lambda b, tok, wgt, be: (0, 0)),
            pl.BlockSpec((1, D, FF), lambda b, tok, wgt, be: (be[b], 0, 0)),
        ],
        out_specs=pl.BlockSpec((T, D), lambda b, tok, wgt, be: (0, 0)),
        scratch_shapes=[pltpu.VMEM((BM, D), f32)],
    )
    out = pl.pallas_call(
        _moe_down_kernel,
        grid_spec=down_spec,
        out_shape=jax.ShapeDtypeStruct((T, D), f32),
    )(dst_tok, dst_w, block_expert, hh, hs, w2)

    return out


# speedup vs baseline: 1.1114x; 1.0584x over previous
import functools

import jax
import jax.numpy as jnp
from jax.experimental import pallas as pl
from jax.experimental.pallas import tpu as pltpu

T = 2048
D = 1024
FF = 2048
H = 16
KV = 8
HD = 64
E = 8
TOPK = 2
EPS = 1e-05
BASE = 1000000.0

BT = 256
BQ = 1024
BM = 128
NPAD = 2 * T + E * BM
NB = NPAD // BM


def _qkv_kernel(x_ref, ln_ref, w_ref, cos_ref, sin_ref, o_ref):
    x = x_ref[...]
    var = jnp.mean(x * x, axis=-1, keepdims=True)
    h = x * jax.lax.rsqrt(var + EPS) * ln_ref[...]
    qkv = jax.lax.dot_general(h, w_ref[...], (((1,), (1,)), ((), ())),
                              preferred_element_type=jnp.float32)
    cos = cos_ref[...]
    sin = sin_ref[...]
    half = HD // 2
    for hd in range(H + KV):
        base = hd * HD
        x1 = qkv[:, base:base + half]
        x2 = qkv[:, base + half:base + HD]
        o_ref[hd, :, :half] = x1 * cos - x2 * sin
        o_ref[hd, :, half:] = x2 * cos + x1 * sin
    for hd in range(H + KV, H + 2 * KV):
        o_ref[hd, :, :] = qkv[:, hd * HD:(hd + 1) * HD]


def _attn_kernel(qoff, kl, q_ref, k_ref, v_ref, o_ref):
    q = q_ref[0]
    k = k_ref[0]
    s = jax.lax.dot_general(q, k, (((1,), (1,)), ((), ())),
                            preferred_element_type=jnp.float32)
    s = s * (HD ** -0.5)
    rows = qoff + jax.lax.broadcasted_iota(jnp.int32, (BQ, kl), 0)
    cols = jax.lax.broadcasted_iota(jnp.int32, (BQ, kl), 1)
    s = jnp.where(rows >= cols, s, -1e30)
    m = jnp.max(s, axis=-1, keepdims=True)
    p = jnp.exp(s - m)
    p = p / jnp.sum(p, axis=-1, keepdims=True)
    o = jax.lax.dot_general(p, v_ref[0], (((1,), (0,)), ((), ())),
                            preferred_element_type=jnp.float32)
    o_ref[...] = o.T


def _post_kernel(o_ref, res_ref, ow_ref, ln_ref, gw_ref,
                 hs_ref, h2_ref, idx_ref, wt_ref):
    attn_out = jax.lax.dot_general(o_ref[...], ow_ref[...],
                                   (((0,), (1,)), ((), ())),
                                   preferred_element_type=jnp.float32)
    hs = res_ref[...] + attn_out
    hs_ref[...] = hs
    var = jnp.mean(hs * hs, axis=-1, keepdims=True)
    h2 = hs * jax.lax.rsqrt(var + EPS) * ln_ref[...]
    h2_ref[...] = h2
    logits = jax.lax.dot_general(h2, gw_ref[...], (((1,), (1,)), ((), ())),
                                 preferred_element_type=jnp.float32)
    lmax = jnp.max(logits, axis=-1, keepdims=True)
    p = jnp.exp(logits - lmax)
    p = p / jnp.sum(p, axis=-1, keepdims=True)
    iota = jax.lax.broadcasted_iota(jnp.int32, (BT, E), 1)
    m1 = jnp.max(p, axis=-1, keepdims=True)
    i1 = jnp.min(jnp.where(p == m1, iota, E), axis=-1, keepdims=True)
    p2 = jnp.where(iota == i1, -1.0, p)
    m2 = jnp.max(p2, axis=-1, keepdims=True)
    i2 = jnp.min(jnp.where(p2 == m2, iota, E), axis=-1, keepdims=True)
    denom = m1 + m2
    idx_ref[...] = jnp.where(iota == 0, i1, jnp.where(iota == 1, i2, 0))
    wt_ref[...] = jnp.where(iota == 0, m1 / denom,
                            jnp.where(iota == 1, m2 / denom, 0.0))


def _moe_up_kernel(tok_ref, be_ref, h2_ref, w1_ref, w3_ref, hh_ref, x_s):
    b = pl.program_id(0)
    base = b * BM

    def gather(i, _):
        t = tok_ref[base + i]
        x_s[pl.ds(i, 1), :] = h2_ref[pl.ds(t, 1), :]
        return 0

    jax.lax.fori_loop(0, BM, gather, 0, unroll=8)

    x = x_s[...]
    h1 = jax.lax.dot_general(x, w1_ref[0], (((1,), (1,)), ((), ())),
                             preferred_element_type=jnp.float32)
    h3 = jax.lax.dot_general(x, w3_ref[0], (((1,), (1,)), ((), ())),
                             preferred_element_type=jnp.float32)
    hh_ref[...] = (h1 * jax.nn.sigmoid(h1)) * h3


def _moe_down_kernel(tok_ref, wgt_ref, be_ref, hh_ref, hs_ref,
                     w2_ref, out_ref, acc_s):
    b = pl.program_id(0)

    @pl.when(b == 0)
    def _init():
        out_ref[...] = hs_ref[...]

    base = b * BM
    acc_s[...] = jax.lax.dot_general(hh_ref[...], w2_ref[0],
                                     (((1,), (1,)), ((), ())),
                                     preferred_element_type=jnp.float32)

    def scatter(i, _):
        t = tok_ref[base + i]
        w = wgt_ref[base + i]
        row = acc_s[pl.ds(i, 1), :]
        out_ref[pl.ds(t, 1), :] = out_ref[pl.ds(t, 1), :] + w * row
        return 0

    jax.lax.fori_loop(0, BM, scatter, 0, unroll=8)


def kernel(positions, hidden_states, ln1_w, qkv_w, o_w, ln2_w, gate_w,
           w1, w2, w3):
    f32 = jnp.float32
    inv_freq = 1.0 / (BASE ** (jnp.arange(0, HD, 2, dtype=f32) / HD))
    f = positions.astype(f32)[:, None] * inv_freq
    cos = jnp.cos(f)
    sin = jnp.sin(f)
    ln1 = ln1_w.reshape(1, D)
    ln2 = ln2_w.reshape(1, D)

    qkv = pl.pallas_call(
        _qkv_kernel,
        grid=(T // BT,),
        in_specs=[
            pl.BlockSpec((BT, D), lambda i: (i, 0)),
            pl.BlockSpec((1, D), lambda i: (0, 0)),
            pl.BlockSpec(((H + 2 * KV) * HD, D), lambda i: (0, 0)),
            pl.BlockSpec((BT, HD // 2), lambda i: (i, 0)),
            pl.BlockSpec((BT, HD // 2), lambda i: (i, 0)),
        ],
        out_specs=pl.BlockSpec((H + 2 * KV, BT, HD), lambda i: (0, i, 0)),
        out_shape=jax.ShapeDtypeStruct((H + 2 * KV, T, HD), f32),
    )(hidden_states, ln1, qkv_w, cos, sin)

    rep = H // KV
    o_parts = []
    for r in range(T // BQ):
        kl = (r + 1) * BQ
        o_parts.append(pl.pallas_call(
            functools.partial(_attn_kernel, r * BQ, kl),
            grid=(H,),
            in_specs=[
                pl.BlockSpec((1, BQ, HD), lambda h, r=r: (h, r, 0)),
                pl.BlockSpec((1, kl, HD), lambda h: (H + h // rep, 0, 0)),
                pl.BlockSpec((1, kl, HD), lambda h: (H + KV + h // rep, 0, 0)),
            ],
            out_specs=pl.BlockSpec((HD, BQ), lambda h: (h, 0)),
            out_shape=jax.ShapeDtypeStruct((H * HD, BQ), f32),
        )(qkv, qkv, qkv))
    o = jnp.concatenate(o_parts, axis=1)

    hs, h2, idx2, wt2 = pl.pallas_call(
        _post_kernel,
        grid=(T // BT,),
        in_specs=[
            pl.BlockSpec((H * HD, BT), lambda i: (0, i)),
            pl.BlockSpec((BT, D), lambda i: (i, 0)),
            pl.BlockSpec((D, H * HD), lambda i: (0, 0)),
            pl.BlockSpec((1, D), lambda i: (0, 0)),
            pl.BlockSpec((E, D), lambda i: (0, 0)),
        ],
        out_specs=[
            pl.BlockSpec((BT, D), lambda i: (i, 0)),
            pl.BlockSpec((BT, D), lambda i: (i, 0)),
            pl.BlockSpec((BT, E), lambda i: (i, 0)),
            pl.BlockSpec((BT, E), lambda i: (i, 0)),
        ],
        out_shape=[
            jax.ShapeDtypeStruct((T, D), f32),
            jax.ShapeDtypeStruct((T, D), f32),
            jax.ShapeDtypeStruct((T, E), jnp.int32),
            jax.ShapeDtypeStruct((T, E), f32),
        ],
    )(o, hidden_states, o_w, ln2, gate_w)

    eflat = idx2[:, :TOPK].reshape(-1)
    tflat = jnp.repeat(jnp.arange(T, dtype=jnp.int32), TOPK).astype(f32)
    wflat = wt2[:, :TOPK].reshape(-1)
    es, ts, ws = jax.lax.sort((eflat, tflat, wflat), num_keys=1,
                              is_stable=False)
    counts = jnp.bincount(eflat, length=E)
    pcounts = ((counts + BM - 1) // BM) * BM
    poff = jnp.concatenate([jnp.zeros((1,), jnp.int32),
                            jnp.cumsum(pcounts).astype(jnp.int32)])
    roff = jnp.cumsum(counts).astype(jnp.int32) - counts.astype(jnp.int32)
    shift = poff[:E] - roff
    pos = jnp.arange(2 * T, dtype=jnp.int32) + shift[es]
    dstp = jnp.zeros((NPAD, 2), f32).at[pos].set(
        jnp.stack([ts, ws], axis=-1))
    dst_tok = dstp[:, 0].astype(jnp.int32)
    dst_w = dstp[:, 1]
    block_expert = jnp.clip(
        jnp.searchsorted(poff[1:], jnp.arange(NB, dtype=jnp.int32) * BM,
                         side='right'), 0, E - 1).astype(jnp.int32)

    up_spec = pltpu.PrefetchScalarGridSpec(
        num_scalar_prefetch=2,
        grid=(NB,),
        in_specs=[
            pl.BlockSpec((T, D), lambda b, tok, be: (0, 0)),
            pl.BlockSpec((1, FF, D), lambda b, tok, be: (be[b], 0, 0)),
            pl.BlockSpec((1, FF, D), lambda b, tok, be: (be[b], 0, 0)),
        ],
        out_specs=pl.BlockSpec((BM, FF), lambda b, tok, be: (b, 0)),
        scratch_shapes=[pltpu.VMEM((BM, D), f32)],
    )
    hh = pl.pallas_call(
        _moe_up_kernel,
        grid_spec=up_spec,
        out_shape=jax.ShapeDtypeStruct((NPAD, FF), f32),
    )(dst_tok, block_expert, h2, w1, w3)

    down_spec = pltpu.PrefetchScalarGridSpec(
        num_scalar_prefetch=3,
        grid=(NB,),
        in_specs=[
            pl.BlockSpec((BM, FF), lambda b, tok, wgt, be: (b, 0)),
            pl.BlockSpec((T, D), lambda b, tok, wgt, be: (0, 0)),
            pl.BlockSpec((1, D, FF), lambda b, tok, wgt, be: (be[b], 0, 0)),
        ],
        out_specs=pl.BlockSpec((T, D), lambda b, tok, wgt, be: (0, 0)),
        scratch_shapes=[pltpu.VMEM((BM, D), f32)],
    )
    out = pl.pallas_call(
        _moe_down_kernel,
        grid_spec=down_spec,
        out_shape=jax.ShapeDtypeStruct((T, D), f32),
    )(dst_tok, dst_w, block_expert, hh, hs, w2)

    return out


# speedup vs baseline: 1.1680x; 1.0509x over previous
import functools

import jax
import jax.numpy as jnp
from jax.experimental import pallas as pl
from jax.experimental.pallas import tpu as pltpu

T = 2048
D = 1024
FF = 2048
H = 16
KV = 8
HD = 64
E = 8
TOPK = 2
EPS = 1e-05
BASE = 1000000.0

BT = 256
BQ = 1024
BM = 128
NPAD = 2 * T + E * BM
NB = NPAD // BM


def _qkv_kernel(x_ref, ln_ref, w_ref, cos_ref, sin_ref, o_ref):
    x = x_ref[...]
    var = jnp.mean(x * x, axis=-1, keepdims=True)
    h = x * jax.lax.rsqrt(var + EPS) * ln_ref[...]
    qkv = jax.lax.dot_general(h, w_ref[...], (((1,), (1,)), ((), ())),
                              preferred_element_type=jnp.float32)
    cos = cos_ref[...]
    sin = sin_ref[...]
    half = HD // 2
    for hd in range(H + KV):
        base = hd * HD
        x1 = qkv[:, base:base + half]
        x2 = qkv[:, base + half:base + HD]
        o_ref[hd, :, :half] = x1 * cos - x2 * sin
        o_ref[hd, :, half:] = x2 * cos + x1 * sin
    for hd in range(H + KV, H + 2 * KV):
        o_ref[hd, :, :] = qkv[:, hd * HD:(hd + 1) * HD]


def _attn_kernel(qoff, kl, q_ref, k_ref, v_ref, o_ref):
    q = q_ref[0]
    k = k_ref[0]
    s = jax.lax.dot_general(q, k, (((1,), (1,)), ((), ())),
                            preferred_element_type=jnp.float32)
    s = s * (HD ** -0.5)
    rows = qoff + jax.lax.broadcasted_iota(jnp.int32, (BQ, kl), 0)
    cols = jax.lax.broadcasted_iota(jnp.int32, (BQ, kl), 1)
    s = jnp.where(rows >= cols, s, -1e30)
    m = jnp.max(s, axis=-1, keepdims=True)
    p = jnp.exp(s - m)
    p = p / jnp.sum(p, axis=-1, keepdims=True)
    o = jax.lax.dot_general(p, v_ref[0], (((1,), (0,)), ((), ())),
                            preferred_element_type=jnp.float32)
    o_ref[...] = o.T


def _post_kernel(o_ref, res_ref, ow_ref, ln_ref, gw_ref,
                 hs_ref, h2_ref, idx_ref, wt_ref):
    attn_out = jax.lax.dot_general(o_ref[...], ow_ref[...],
                                   (((0,), (1,)), ((), ())),
                                   preferred_element_type=jnp.float32)
    hs = res_ref[...] + attn_out
    hs_ref[...] = hs
    var = jnp.mean(hs * hs, axis=-1, keepdims=True)
    h2 = hs * jax.lax.rsqrt(var + EPS) * ln_ref[...]
    h2_ref[...] = h2
    logits = jax.lax.dot_general(h2, gw_ref[...], (((1,), (1,)), ((), ())),
                                 preferred_element_type=jnp.float32)
    lmax = jnp.max(logits, axis=-1, keepdims=True)
    p = jnp.exp(logits - lmax)
    p = p / jnp.sum(p, axis=-1, keepdims=True)
    iota = jax.lax.broadcasted_iota(jnp.int32, (BT, E), 1)
    m1 = jnp.max(p, axis=-1, keepdims=True)
    i1 = jnp.min(jnp.where(p == m1, iota, E), axis=-1, keepdims=True)
    p2 = jnp.where(iota == i1, -1.0, p)
    m2 = jnp.max(p2, axis=-1, keepdims=True)
    i2 = jnp.min(jnp.where(p2 == m2, iota, E), axis=-1, keepdims=True)
    denom = m1 + m2
    idx_ref[...] = jnp.where(iota == 0, i1, jnp.where(iota == 1, i2, 0))
    wt_ref[...] = jnp.where(iota == 0, m1 / denom,
                            jnp.where(iota == 1, m2 / denom, 0.0))


def _moe_up_kernel(tok_ref, be_ref, nb_ref, h2_ref, w1_ref, w3_ref,
                   hh_ref, x_s):
    b = pl.program_id(0)

    @pl.when(b < nb_ref[0])
    def _body():
        base = b * BM

        def gather(i, _):
            t = tok_ref[base + i]
            x_s[pl.ds(i, 1), :] = h2_ref[pl.ds(t, 1), :]
            return 0

        jax.lax.fori_loop(0, BM, gather, 0, unroll=8)

        x = x_s[...]
        h1 = jax.lax.dot_general(x, w1_ref[0], (((1,), (1,)), ((), ())),
                                 preferred_element_type=jnp.float32)
        h3 = jax.lax.dot_general(x, w3_ref[0], (((1,), (1,)), ((), ())),
                                 preferred_element_type=jnp.float32)
        hh_ref[...] = (h1 * jax.nn.sigmoid(h1)) * h3


def _moe_down_kernel(tok_ref, wgt_ref, be_ref, nb_ref, hh_ref, hs_ref,
                     w2_ref, out_ref, acc_s):
    b = pl.program_id(0)

    @pl.when(b == 0)
    def _init():
        out_ref[...] = hs_ref[...]

    @pl.when(b < nb_ref[0])
    def _body():
        base = b * BM
        acc_s[...] = jax.lax.dot_general(hh_ref[...], w2_ref[0],
                                         (((1,), (1,)), ((), ())),
                                         preferred_element_type=jnp.float32)

        def scatter(i, _):
            t = tok_ref[base + i]
            w = wgt_ref[base + i]
            row = acc_s[pl.ds(i, 1), :]
            out_ref[pl.ds(t, 1), :] = out_ref[pl.ds(t, 1), :] + w * row
            return 0

        jax.lax.fori_loop(0, BM, scatter, 0, unroll=8)


def kernel(positions, hidden_states, ln1_w, qkv_w, o_w, ln2_w, gate_w,
           w1, w2, w3):
    f32 = jnp.float32
    inv_freq = 1.0 / (BASE ** (jnp.arange(0, HD, 2, dtype=f32) / HD))
    f = positions.astype(f32)[:, None] * inv_freq
    cos = jnp.cos(f)
    sin = jnp.sin(f)
    ln1 = ln1_w.reshape(1, D)
    ln2 = ln2_w.reshape(1, D)

    qkv = pl.pallas_call(
        _qkv_kernel,
        grid=(T // BT,),
        in_specs=[
            pl.BlockSpec((BT, D), lambda i: (i, 0)),
            pl.BlockSpec((1, D), lambda i: (0, 0)),
            pl.BlockSpec(((H + 2 * KV) * HD, D), lambda i: (0, 0)),
            pl.BlockSpec((BT, HD // 2), lambda i: (i, 0)),
            pl.BlockSpec((BT, HD // 2), lambda i: (i, 0)),
        ],
        out_specs=pl.BlockSpec((H + 2 * KV, BT, HD), lambda i: (0, i, 0)),
        out_shape=jax.ShapeDtypeStruct((H + 2 * KV, T, HD), f32),
    )(hidden_states, ln1, qkv_w, cos, sin)

    rep = H // KV
    o_parts = []
    for r in range(T // BQ):
        kl = (r + 1) * BQ
        o_parts.append(pl.pallas_call(
            functools.partial(_attn_kernel, r * BQ, kl),
            grid=(H,),
            in_specs=[
                pl.BlockSpec((1, BQ, HD), lambda h, r=r: (h, r, 0)),
                pl.BlockSpec((1, kl, HD), lambda h: (H + h // rep, 0, 0)),
                pl.BlockSpec((1, kl, HD), lambda h: (H + KV + h // rep, 0, 0)),
            ],
            out_specs=pl.BlockSpec((HD, BQ), lambda h: (h, 0)),
            out_shape=jax.ShapeDtypeStruct((H * HD, BQ), f32),
        )(qkv, qkv, qkv))
    o = jnp.concatenate(o_parts, axis=1)

    hs, h2, idx2, wt2 = pl.pallas_call(
        _post_kernel,
        grid=(T // BT,),
        in_specs=[
            pl.BlockSpec((H * HD, BT), lambda i: (0, i)),
            pl.BlockSpec((BT, D), lambda i: (i, 0)),
            pl.BlockSpec((D, H * HD), lambda i: (0, 0)),
            pl.BlockSpec((1, D), lambda i: (0, 0)),
            pl.BlockSpec((E, D), lambda i: (0, 0)),
        ],
        out_specs=[
            pl.BlockSpec((BT, D), lambda i: (i, 0)),
            pl.BlockSpec((BT, D), lambda i: (i, 0)),
            pl.BlockSpec((BT, E), lambda i: (i, 0)),
            pl.BlockSpec((BT, E), lambda i: (i, 0)),
        ],
        out_shape=[
            jax.ShapeDtypeStruct((T, D), f32),
            jax.ShapeDtypeStruct((T, D), f32),
            jax.ShapeDtypeStruct((T, E), jnp.int32),
            jax.ShapeDtypeStruct((T, E), f32),
        ],
    )(o, hidden_states, o_w, ln2, gate_w)

    eflat = idx2[:, :TOPK].reshape(-1)
    tflat = jnp.repeat(jnp.arange(T, dtype=jnp.int32), TOPK).astype(f32)
    wflat = wt2[:, :TOPK].reshape(-1)
    es, ts, ws = jax.lax.sort((eflat, tflat, wflat), num_keys=1,
                              is_stable=False)
    counts = jnp.bincount(eflat, length=E)
    pcounts = ((counts + BM - 1) // BM) * BM
    poff = jnp.concatenate([jnp.zeros((1,), jnp.int32),
                            jnp.cumsum(pcounts).astype(jnp.int32)])
    roff = jnp.cumsum(counts).astype(jnp.int32) - counts.astype(jnp.int32)
    shift = poff[:E] - roff
    pos = jnp.arange(2 * T, dtype=jnp.int32) + shift[es]
    dstp = jnp.zeros((NPAD, 2), f32).at[pos].set(
        jnp.stack([ts, ws], axis=-1))
    dst_tok = dstp[:, 0].astype(jnp.int32)
    dst_w = dstp[:, 1]
    block_expert = jnp.clip(
        jnp.searchsorted(poff[1:], jnp.arange(NB, dtype=jnp.int32) * BM,
                         side='right'), 0, E - 1).astype(jnp.int32)

    nblk = (poff[E:] // BM).astype(jnp.int32)
    up_spec = pltpu.PrefetchScalarGridSpec(
        num_scalar_prefetch=3,
        grid=(NB,),
        in_specs=[
            pl.BlockSpec((T, D), lambda b, tok, be, nb: (0, 0)),
            pl.BlockSpec((1, FF, D), lambda b, tok, be, nb: (be[b], 0, 0)),
            pl.BlockSpec((1, FF, D), lambda b, tok, be, nb: (be[b], 0, 0)),
        ],
        out_specs=pl.BlockSpec((BM, FF), lambda b, tok, be, nb: (b, 0)),
        scratch_shapes=[pltpu.VMEM((BM, D), f32)],
    )
    hh = pl.pallas_call(
        _moe_up_kernel,
        grid_spec=up_spec,
        out_shape=jax.ShapeDtypeStruct((NPAD, FF), f32),
    )(dst_tok, block_expert, nblk, h2, w1, w3)

    down_spec = pltpu.PrefetchScalarGridSpec(
        num_scalar_prefetch=4,
        grid=(NB,),
        in_specs=[
            pl.BlockSpec((BM, FF), lambda b, tok, wgt, be, nb: (b, 0)),
            pl.BlockSpec((T, D), lambda b, tok, wgt, be, nb: (0, 0)),
            pl.BlockSpec((1, D, FF),
                         lambda b, tok, wgt, be, nb: (be[b], 0, 0)),
        ],
        out_specs=pl.BlockSpec((T, D), lambda b, tok, wgt, be, nb: (0, 0)),
        scratch_shapes=[pltpu.VMEM((BM, D), f32)],
    )
    out = pl.pallas_call(
        _moe_down_kernel,
        grid_spec=down_spec,
        out_shape=jax.ShapeDtypeStruct((T, D), f32),
    )(dst_tok, dst_w, block_expert, nblk, hh, hs, w2)

    return out


# speedup vs baseline: 1.2032x; 1.0301x over previous
import functools

import jax
import jax.numpy as jnp
from jax.experimental import pallas as pl
from jax.experimental.pallas import tpu as pltpu

T = 2048
D = 1024
FF = 2048
H = 16
KV = 8
HD = 64
E = 8
TOPK = 2
EPS = 1e-05
BASE = 1000000.0

BT = 256
BQ = 1024
BM = 128
NPAD = 2 * T + E * BM
NB = NPAD // BM


def _qkv_kernel(x_ref, ln_ref, w_ref, cos_ref, sin_ref, o_ref):
    x = x_ref[...]
    var = jnp.mean(x * x, axis=-1, keepdims=True)
    h = x * jax.lax.rsqrt(var + EPS) * ln_ref[...]
    qkv = jax.lax.dot_general(h, w_ref[...], (((1,), (1,)), ((), ())),
                              preferred_element_type=jnp.float32)
    cos = cos_ref[...]
    sin = sin_ref[...]
    half = HD // 2
    for hd in range(H + KV):
        base = hd * HD
        x1 = qkv[:, base:base + half]
        x2 = qkv[:, base + half:base + HD]
        o_ref[hd, :, :half] = x1 * cos - x2 * sin
        o_ref[hd, :, half:] = x2 * cos + x1 * sin
    for hd in range(H + KV, H + 2 * KV):
        o_ref[hd, :, :] = qkv[:, hd * HD:(hd + 1) * HD]


def _attn_kernel(qoff, kl, q_ref, k_ref, v_ref, o_ref):
    q = q_ref[0]
    k = k_ref[0]
    s = jax.lax.dot_general(q, k, (((1,), (1,)), ((), ())),
                            preferred_element_type=jnp.float32)
    s = s * (HD ** -0.5)
    rows = qoff + jax.lax.broadcasted_iota(jnp.int32, (BQ, kl), 0)
    cols = jax.lax.broadcasted_iota(jnp.int32, (BQ, kl), 1)
    s = jnp.where(rows >= cols, s, -1e30)
    m = jnp.max(s, axis=-1, keepdims=True)
    p = jnp.exp(s - m)
    p = p / jnp.sum(p, axis=-1, keepdims=True)
    o = jax.lax.dot_general(p, v_ref[0], (((1,), (0,)), ((), ())),
                            preferred_element_type=jnp.float32)
    o_ref[...] = o.T


def _post_kernel(o_ref, res_ref, ow_ref, ln_ref, gw_ref,
                 hs_ref, h2_ref, idx_ref, wt_ref):
    attn_out = jax.lax.dot_general(o_ref[...], ow_ref[...],
                                   (((0,), (1,)), ((), ())),
                                   preferred_element_type=jnp.float32)
    hs = res_ref[...] + attn_out
    hs_ref[...] = hs
    var = jnp.mean(hs * hs, axis=-1, keepdims=True)
    h2 = hs * jax.lax.rsqrt(var + EPS) * ln_ref[...]
    h2_ref[...] = h2
    logits = jax.lax.dot_general(h2, gw_ref[...], (((1,), (1,)), ((), ())),
                                 preferred_element_type=jnp.float32)
    lmax = jnp.max(logits, axis=-1, keepdims=True)
    p = jnp.exp(logits - lmax)
    p = p / jnp.sum(p, axis=-1, keepdims=True)
    iota = jax.lax.broadcasted_iota(jnp.int32, (BT, E), 1)
    m1 = jnp.max(p, axis=-1, keepdims=True)
    i1 = jnp.min(jnp.where(p == m1, iota, E), axis=-1, keepdims=True)
    p2 = jnp.where(iota == i1, -1.0, p)
    m2 = jnp.max(p2, axis=-1, keepdims=True)
    i2 = jnp.min(jnp.where(p2 == m2, iota, E), axis=-1, keepdims=True)
    denom = m1 + m2
    idx_ref[...] = jnp.where(iota == 0, i1, jnp.where(iota == 1, i2, 0))
    wt_ref[...] = jnp.where(iota == 0, m1 / denom,
                            jnp.where(iota == 1, m2 / denom, 0.0))


def _moe_up_kernel(tok_ref, be_ref, nb_ref, h2_ref, w1_ref, w3_ref,
                   hh_ref, x_s):
    b = pl.program_id(0)

    @pl.when(b < nb_ref[0])
    def _body():
        base = b * BM

        def gather(i, _):
            t = tok_ref[base + i]
            x_s[pl.ds(i, 1), :] = h2_ref[pl.ds(t, 1), :]
            return 0

        jax.lax.fori_loop(0, BM, gather, 0, unroll=8)

        x = x_s[...]
        h1 = jax.lax.dot_general(x, w1_ref[0], (((1,), (1,)), ((), ())),
                                 preferred_element_type=jnp.float32)
        h3 = jax.lax.dot_general(x, w3_ref[0], (((1,), (1,)), ((), ())),
                                 preferred_element_type=jnp.float32)
        hh_ref[...] = (h1 * jax.nn.sigmoid(h1)) * h3


def _moe_down_kernel(tok_ref, wgt_ref, be_ref, nb_ref, hh_ref, hs_ref,
                     w2_ref, out_ref, acc_s):
    b = pl.program_id(0)

    @pl.when(b == 0)
    def _init():
        out_ref[...] = hs_ref[...]

    @pl.when(b < nb_ref[0])
    def _body():
        base = b * BM
        acc_s[...] = jax.lax.dot_general(hh_ref[...], w2_ref[0],
                                         (((1,), (1,)), ((), ())),
                                         preferred_element_type=jnp.float32)

        def scatter(i, _):
            t = tok_ref[base + i]
            w = wgt_ref[base + i]
            row = acc_s[pl.ds(i, 1), :]
            out_ref[pl.ds(t, 1), :] = out_ref[pl.ds(t, 1), :] + w * row
            return 0

        jax.lax.fori_loop(0, BM, scatter, 0, unroll=8)


def kernel(positions, hidden_states, ln1_w, qkv_w, o_w, ln2_w, gate_w,
           w1, w2, w3):
    f32 = jnp.float32
    inv_freq = 1.0 / (BASE ** (jnp.arange(0, HD, 2, dtype=f32) / HD))
    f = positions.astype(f32)[:, None] * inv_freq
    cos = jnp.cos(f)
    sin = jnp.sin(f)
    ln1 = ln1_w.reshape(1, D)
    ln2 = ln2_w.reshape(1, D)

    qkv = pl.pallas_call(
        _qkv_kernel,
        grid=(T // BT,),
        in_specs=[
            pl.BlockSpec((BT, D), lambda i: (i, 0)),
            pl.BlockSpec((1, D), lambda i: (0, 0)),
            pl.BlockSpec(((H + 2 * KV) * HD, D), lambda i: (0, 0)),
            pl.BlockSpec((BT, HD // 2), lambda i: (i, 0)),
            pl.BlockSpec((BT, HD // 2), lambda i: (i, 0)),
        ],
        out_specs=pl.BlockSpec((H + 2 * KV, BT, HD), lambda i: (0, i, 0)),
        out_shape=jax.ShapeDtypeStruct((H + 2 * KV, T, HD), f32),
    )(hidden_states, ln1, qkv_w, cos, sin)

    rep = H // KV
    o_parts = []
    for r in range(T // BQ):
        kl = (r + 1) * BQ
        o_parts.append(pl.pallas_call(
            functools.partial(_attn_kernel, r * BQ, kl),
            grid=(H,),
            in_specs=[
                pl.BlockSpec((1, BQ, HD), lambda h, r=r: (h, r, 0)),
                pl.BlockSpec((1, kl, HD), lambda h: (H + h // rep, 0, 0)),
                pl.BlockSpec((1, kl, HD), lambda h: (H + KV + h // rep, 0, 0)),
            ],
            out_specs=pl.BlockSpec((HD, BQ), lambda h: (h, 0)),
            out_shape=jax.ShapeDtypeStruct((H * HD, BQ), f32),
        )(qkv, qkv, qkv))
    o = jnp.concatenate(o_parts, axis=1)

    hs, h2, idx2, wt2 = pl.pallas_call(
        _post_kernel,
        grid=(T // BT,),
        in_specs=[
            pl.BlockSpec((H * HD, BT), lambda i: (0, i)),
            pl.BlockSpec((BT, D), lambda i: (i, 0)),
            pl.BlockSpec((D, H * HD), lambda i: (0, 0)),
            pl.BlockSpec((1, D), lambda i: (0, 0)),
            pl.BlockSpec((E, D), lambda i: (0, 0)),
        ],
        out_specs=[
            pl.BlockSpec((BT, D), lambda i: (i, 0)),
            pl.BlockSpec((BT, D), lambda i: (i, 0)),
            pl.BlockSpec((BT, E), lambda i: (i, 0)),
            pl.BlockSpec((BT, E), lambda i: (i, 0)),
        ],
        out_shape=[
            jax.ShapeDtypeStruct((T, D), f32),
            jax.ShapeDtypeStruct((T, D), f32),
            jax.ShapeDtypeStruct((T, E), jnp.int32),
            jax.ShapeDtypeStruct((T, E), f32),
        ],
    )(o, hidden_states, o_w, ln2, gate_w)

    eflat = idx2[:, :TOPK].reshape(-1)
    tflat = jnp.repeat(jnp.arange(T, dtype=jnp.int32), TOPK).astype(f32)
    wflat = wt2[:, :TOPK].reshape(-1)
    es, ts, ws = jax.lax.sort((eflat, tflat, wflat), num_keys=1,
                              is_stable=False)
    bounds = jnp.searchsorted(es, jnp.arange(E + 1, dtype=jnp.int32))
    counts = (bounds[1:] - bounds[:-1]).astype(jnp.int32)
    pcounts = ((counts + BM - 1) // BM) * BM
    poff = jnp.concatenate([jnp.zeros((1,), jnp.int32),
                            jnp.cumsum(pcounts).astype(jnp.int32)])
    roff = jnp.cumsum(counts).astype(jnp.int32) - counts.astype(jnp.int32)
    shift = poff[:E] - roff
    pos = jnp.arange(2 * T, dtype=jnp.int32) + shift[es]
    dstp = jnp.zeros((NPAD, 2), f32).at[pos].set(
        jnp.stack([ts, ws], axis=-1))
    dst_tok = dstp[:, 0].astype(jnp.int32)
    dst_w = dstp[:, 1]
    block_expert = jnp.clip(
        jnp.searchsorted(poff[1:], jnp.arange(NB, dtype=jnp.int32) * BM,
                         side='right'), 0, E - 1).astype(jnp.int32)

    nblk = (poff[E:] // BM).astype(jnp.int32)
    up_spec = pltpu.PrefetchScalarGridSpec(
        num_scalar_prefetch=3,
        grid=(NB,),
        in_specs=[
            pl.BlockSpec((T, D), lambda b, tok, be, nb: (0, 0)),
            pl.BlockSpec((1, FF, D), lambda b, tok, be, nb: (be[b], 0, 0)),
            pl.BlockSpec((1, FF, D), lambda b, tok, be, nb: (be[b], 0, 0)),
        ],
        out_specs=pl.BlockSpec((BM, FF), lambda b, tok, be, nb: (b, 0)),
        scratch_shapes=[pltpu.VMEM((BM, D), f32)],
    )
    hh = pl.pallas_call(
        _moe_up_kernel,
        grid_spec=up_spec,
        out_shape=jax.ShapeDtypeStruct((NPAD, FF), f32),
    )(dst_tok, block_expert, nblk, h2, w1, w3)

    down_spec = pltpu.PrefetchScalarGridSpec(
        num_scalar_prefetch=4,
        grid=(NB,),
        in_specs=[
            pl.BlockSpec((BM, FF), lambda b, tok, wgt, be, nb: (b, 0)),
            pl.BlockSpec((T, D), lambda b, tok, wgt, be, nb: (0, 0)),
            pl.BlockSpec((1, D, FF),
                         lambda b, tok, wgt, be, nb: (be[b], 0, 0)),
        ],
        out_specs=pl.BlockSpec((T, D), lambda b, tok, wgt, be, nb: (0, 0)),
        scratch_shapes=[pltpu.VMEM((BM, D), f32)],
    )
    out = pl.pallas_call(
        _moe_down_kernel,
        grid_spec=down_spec,
        out_shape=jax.ShapeDtypeStruct((T, D), f32),
    )(dst_tok, dst_w, block_expert, nblk, hh, hs, w2)

    return out
